# Initial kernel scaffold; baseline (speedup 1.0000x reference)
import jax
import jax.numpy as jnp
from jax.experimental import pallas as pl


def kernel(v_pred, x0, x1, fixed, batch_idx, num_systems):
    raise NotImplementedError("write your pallas kernel here")


# speedup vs baseline: 3.5530x; 3.5530x over previous
import functools

import jax
import jax.numpy as jnp
from jax import lax
from jax.experimental import pallas as pl
from jax.experimental.pallas import tpu as pltpu
from jax.experimental.pallas import tpu_sc as plsc

_S = 8192
_NC = 2
_NS = 16
_NW = _NC * _NS
_L = 16
_BLOCK_A = 2048


def _sc_partials(vp_hbm, x0_hbm, x1_hbm, w_hbm, bi_hbm,
                 part_hbm, ss_hbm,
                 vp_b, x0_b, x1_b, w_b, bi_b, acc, ssbuf,
                 *, atoms_per_tile):
    wid = lax.axis_index("s") * _NC + lax.axis_index("c")
    base_atom = wid * atoms_per_tile
    nblk = atoms_per_tile // _BLOCK_A

    zero16 = jnp.zeros((_L,), jnp.float32)

    def zbody(i, c):
        acc[pl.ds(i * _L, _L)] = zero16
        return c

    lax.fori_loop(0, (5 * _S) // _L, zbody, 0)

    lane = lax.iota(jnp.int32, _L)
    ss = jnp.zeros((_L,), jnp.float32)

    for blk in range(nblk):
        a0 = base_atom + blk * _BLOCK_A
        pltpu.sync_copy(vp_hbm.at[pl.ds(3 * a0, 3 * _BLOCK_A)], vp_b)
        pltpu.sync_copy(x0_hbm.at[pl.ds(3 * a0, 3 * _BLOCK_A)], x0_b)
        pltpu.sync_copy(x1_hbm.at[pl.ds(3 * a0, 3 * _BLOCK_A)], x1_b)
        pltpu.sync_copy(w_hbm.at[pl.ds(a0, _BLOCK_A)], w_b)
        pltpu.sync_copy(bi_hbm.at[pl.ds(a0, _BLOCK_A)], bi_b)

        def gbody(g2, ss):
            for k in range(3):
                j0 = g2 * 48 + k * _L
                fl = k * _L + lane
                arel = (fl * 21846) >> 16
                jm = fl - arel * 3
                vp = vp_b[pl.ds(j0, _L)]
                x0v = x0_b[pl.ds(j0, _L)]
                x1v = x1_b[pl.ds(j0, _L)]
                d = vp - x1v + x0v
                aidx = g2 * _L + arel
                wv = plsc.load_gather(w_b, [aidx])
                bv = plsc.load_gather(bi_b, [aidx])
                wd = wv * d
                ss = ss + wd * d
                plsc.addupdate_scatter(acc, [jm * _S + bv], wd)
            ag = g2 * _L
            wg = w_b[pl.ds(ag, _L)]
            bg = bi_b[pl.ds(ag, _L)]
            plsc.addupdate_scatter(acc, [3 * _S + bg], wg)
            plsc.addupdate_scatter(acc, [4 * _S + bg], 1.0 - wg)
            return ss

        ss = lax.fori_loop(0, _BLOCK_A // _L, gbody, ss)

    ssbuf[...] = ss
    pltpu.sync_copy(acc, part_hbm.at[wid])
    pltpu.sync_copy(ssbuf, ss_hbm.at[wid])


def _tc_merge(part_ref, ss_ref, out_ref):
    p = part_ref[...]
    m = jnp.sum(p, axis=0)
    sx = m[0:1]
    sy = m[1:2]
    sz = m[2:3]
    cm = m[3:4]
    cf = m[4:5]
    s2 = sx * sx + sy * sy + sz * sz
    corr = jnp.sum(jnp.where(cf == 0.0, s2 / jnp.maximum(cm, 1.0), 0.0))
    nm = jnp.sum(cm)
    ssq = jnp.sum(ss_ref[...])
    out_ref[0, 0] = (ssq - corr) / jnp.maximum(nm, 1.0)


def kernel(v_pred, x0, x1, fixed, batch_idx, num_systems):
    n = v_pred.shape[0]
    atoms_per_tile = n // _NW

    vp_flat = v_pred.reshape(-1)
    x0_flat = x0.reshape(-1)
    x1_flat = x1.reshape(-1)
    w = (~fixed).astype(jnp.float32)
    bi = batch_idx.astype(jnp.int32)

    mesh = plsc.VectorSubcoreMesh(core_axis_name="c", subcore_axis_name="s")
    part, ss = pl.kernel(
        functools.partial(_sc_partials, atoms_per_tile=atoms_per_tile),
        out_type=(
            jax.ShapeDtypeStruct((_NW, 5 * _S), jnp.float32),
            jax.ShapeDtypeStruct((_NW, _L), jnp.float32),
        ),
        mesh=mesh,
        compiler_params=pltpu.CompilerParams(needs_layout_passes=False),
        scratch_types=(
            pltpu.VMEM((3 * _BLOCK_A,), jnp.float32),
            pltpu.VMEM((3 * _BLOCK_A,), jnp.float32),
            pltpu.VMEM((3 * _BLOCK_A,), jnp.float32),
            pltpu.VMEM((_BLOCK_A,), jnp.float32),
            pltpu.VMEM((_BLOCK_A,), jnp.int32),
            pltpu.VMEM((5 * _S,), jnp.float32),
            pltpu.VMEM((_L,), jnp.float32),
        ),
    )(vp_flat, x0_flat, x1_flat, w, bi)

    out = pl.pallas_call(
        _tc_merge,
        out_shape=jax.ShapeDtypeStruct((1, 1), jnp.float32),
        out_specs=pl.BlockSpec(memory_space=pltpu.SMEM),
    )(part.reshape(_NW, 5, _S), ss)

    loss = out[0, 0]
    return loss + jnp.zeros_like(loss) * num_systems


# speedup vs baseline: 31.2166x; 8.7861x over previous
import functools

import jax
import jax.numpy as jnp
from jax import lax
from jax.experimental import pallas as pl
from jax.experimental.pallas import tpu as pltpu
from jax.experimental.pallas import tpu_sc as plsc

_S = 8192
_NC = 2
_NS = 16
_NW = _NC * _NS
_L = 16
_BLOCK_A = 2048


def _sc_partials(vx_h, vy_h, vz_h, ax_h, ay_h, az_h, bx_h, by_h, bz_h,
                 w_h, bi_h,
                 part_hbm, ss_hbm,
                 b0, b1, b2, b3, b4, b5, b6, b7, b8, b9, b10, acc, ssbuf,
                 *, atoms_per_tile):
    wid = lax.axis_index("s") * _NC + lax.axis_index("c")
    base_atom = wid * atoms_per_tile
    nblk = atoms_per_tile // _BLOCK_A
    ins = (vx_h, vy_h, vz_h, ax_h, ay_h, az_h, bx_h, by_h, bz_h, w_h, bi_h)
    bufs = (b0, b1, b2, b3, b4, b5, b6, b7, b8, b9, b10)

    zero16 = jnp.zeros((_L,), jnp.float32)

    def zbody(i, c):
        acc[pl.ds(i * _L, _L)] = zero16
        return c

    lax.fori_loop(0, (5 * _S) // _L, zbody, 0)

    ss = jnp.zeros((_L,), jnp.float32)

    for blk in range(nblk):
        a0 = base_atom + blk * _BLOCK_A
        for h, b in zip(ins, bufs):
            pltpu.sync_copy(h.at[pl.ds(a0, _BLOCK_A)], b)

        def gbody(g, ss):
            o = g * _L
            dx = b0[pl.ds(o, _L)] - b6[pl.ds(o, _L)] + b3[pl.ds(o, _L)]
            dy = b1[pl.ds(o, _L)] - b7[pl.ds(o, _L)] + b4[pl.ds(o, _L)]
            dz = b2[pl.ds(o, _L)] - b8[pl.ds(o, _L)] + b5[pl.ds(o, _L)]
            wv = b9[pl.ds(o, _L)]
            bv = b10[pl.ds(o, _L)]
            wdx = wv * dx
            wdy = wv * dy
            wdz = wv * dz
            ss = ss + wdx * dx + wdy * dy + wdz * dz
            plsc.addupdate_scatter(acc, [bv], wdx)
            plsc.addupdate_scatter(acc, [_S + bv], wdy)
            plsc.addupdate_scatter(acc, [2 * _S + bv], wdz)
            plsc.addupdate_scatter(acc, [3 * _S + bv], wv)
            plsc.addupdate_scatter(acc, [4 * _S + bv], 1.0 - wv)
            return ss

        ss = lax.fori_loop(0, _BLOCK_A // _L, gbody, ss)

    ssbuf[...] = ss
    pltpu.sync_copy(acc, part_hbm.at[wid])
    pltpu.sync_copy(ssbuf, ss_hbm.at[wid])


def _tc_merge(part_ref, ss_ref, out_ref):
    p = part_ref[...]
    m = jnp.sum(p, axis=0)
    sx = m[0:1]
    sy = m[1:2]
    sz = m[2:3]
    cm = m[3:4]
    cf = m[4:5]
    s2 = sx * sx + sy * sy + sz * sz
    corr = jnp.sum(jnp.where(cf == 0.0, s2 / jnp.maximum(cm, 1.0), 0.0))
    nm = jnp.sum(cm)
    ssq = jnp.sum(ss_ref[...])
    out_ref[0, 0] = (ssq - corr) / jnp.maximum(nm, 1.0)


def kernel(v_pred, x0, x1, fixed, batch_idx, num_systems):
    n = v_pred.shape[0]
    atoms_per_tile = n // _NW

    planes = [v_pred[:, c] for c in range(3)]
    planes += [x0[:, c] for c in range(3)]
    planes += [x1[:, c] for c in range(3)]
    w = (~fixed).astype(jnp.float32)
    bi = batch_idx.astype(jnp.int32)

    mesh = plsc.VectorSubcoreMesh(core_axis_name="c", subcore_axis_name="s")
    part, ss = pl.kernel(
        functools.partial(_sc_partials, atoms_per_tile=atoms_per_tile),
        out_type=(
            jax.ShapeDtypeStruct((_NW, 5 * _S), jnp.float32),
            jax.ShapeDtypeStruct((_NW, _L), jnp.float32),
        ),
        mesh=mesh,
        compiler_params=pltpu.CompilerParams(needs_layout_passes=False),
        scratch_types=(
            *[pltpu.VMEM((_BLOCK_A,), jnp.float32) for _ in range(10)],
            pltpu.VMEM((_BLOCK_A,), jnp.int32),
            pltpu.VMEM((5 * _S,), jnp.float32),
            pltpu.VMEM((_L,), jnp.float32),
        ),
    )(*planes, w, bi)

    out = pl.pallas_call(
        _tc_merge,
        out_shape=jax.ShapeDtypeStruct((1, 1), jnp.float32),
        out_specs=pl.BlockSpec(memory_space=pltpu.SMEM),
    )(part.reshape(_NW, 5, _S), ss)

    loss = out[0, 0]
    return loss + jnp.zeros_like(loss) * num_systems


# speedup vs baseline: 68.9419x; 2.2085x over previous
import functools

import jax
import jax.numpy as jnp
from jax import lax
from jax.experimental import pallas as pl
from jax.experimental.pallas import tpu as pltpu
from jax.experimental.pallas import tpu_sc as plsc

_S = 8192
_NC = 2
_NS = 16
_NW = _NC * _NS
_L = 16
_BLOCK_A = 2048


def _sc_partials(vp_h, x0_h, x1_h, w_h, bi_h,
                 part_hbm, ss_hbm,
                 bufs0, bufs1, sem0, sem1, acc, ssbuf,
                 *, atoms_per_tile, n):
    wid = lax.axis_index("s") * _NC + lax.axis_index("c")
    base_atom = wid * atoms_per_tile
    nblk = atoms_per_tile // _BLOCK_A
    sems = (sem0, sem1)

    zero16 = jnp.zeros((_L,), jnp.float32)

    def zbody(i, c):
        o = i * 4 * _L
        acc[pl.ds(o, _L)] = zero16
        acc[pl.ds(o + _L, _L)] = zero16
        acc[pl.ds(o + 2 * _L, _L)] = zero16
        acc[pl.ds(o + 3 * _L, _L)] = zero16
        return c

    lax.fori_loop(0, (5 * _S) // (4 * _L), zbody, 0)

    def start_block(blk, parity):
        a0 = base_atom + blk * _BLOCK_A
        bufs, sem = (bufs0, bufs1)[parity], sems[parity]
        hs = []
        for c in range(3):
            hs.append(pltpu.async_copy(
                vp_h.at[pl.ds(c * n + a0, _BLOCK_A)], bufs[c], sem))
            hs.append(pltpu.async_copy(
                x0_h.at[pl.ds(c * n + a0, _BLOCK_A)], bufs[3 + c], sem))
            hs.append(pltpu.async_copy(
                x1_h.at[pl.ds(c * n + a0, _BLOCK_A)], bufs[6 + c], sem))
        hs.append(pltpu.async_copy(w_h.at[pl.ds(a0, _BLOCK_A)], bufs[9], sem))
        hs.append(pltpu.async_copy(bi_h.at[pl.ds(a0, _BLOCK_A)], bufs[10], sem))
        return hs

    pending = {0: start_block(0, 0)}

    ss = jnp.zeros((_L,), jnp.float32)
    curb = jnp.full((_L,), -1, jnp.int32)
    rsx = zero16
    rsy = zero16
    rsz = zero16
    rcm = zero16
    rcf = zero16
    carry = (ss, curb, rsx, rsy, rsz, rcm, rcf)

    for blk in range(nblk):
        parity = blk % 2
        if blk + 1 < nblk:
            pending[(blk + 1) % 2] = start_block(blk + 1, (blk + 1) % 2)
        for h in pending.pop(parity):
            h.wait()
        bufs = (bufs0, bufs1)[parity]
        b0, b1, b2, b3, b4, b5, b6, b7, b8, b9, b10 = bufs

        def gbody(g, carry):
            ss, curb, rsx, rsy, rsz, rcm, rcf = carry
            o = g * _L
            dx = b0[pl.ds(o, _L)] - b6[pl.ds(o, _L)] + b3[pl.ds(o, _L)]
            dy = b1[pl.ds(o, _L)] - b7[pl.ds(o, _L)] + b4[pl.ds(o, _L)]
            dz = b2[pl.ds(o, _L)] - b8[pl.ds(o, _L)] + b5[pl.ds(o, _L)]
            wv = b9[pl.ds(o, _L)]
            bv = b10[pl.ds(o, _L)]
            wdx = wv * dx
            wdy = wv * dy
            wdz = wv * dz
            ss = ss + wdx * dx + wdy * dy + wdz * dz
            same = bv == curb
            flush = jnp.logical_not(same) & (curb >= 0)
            ci = jnp.maximum(curb, 0)
            plsc.addupdate_scatter(acc, [ci], rsx, mask=flush)
            plsc.addupdate_scatter(acc, [_S + ci], rsy, mask=flush)
            plsc.addupdate_scatter(acc, [2 * _S + ci], rsz, mask=flush)
            plsc.addupdate_scatter(acc, [3 * _S + ci], rcm, mask=flush)
            plsc.addupdate_scatter(acc, [4 * _S + ci], rcf, mask=flush)
            rsx = jnp.where(same, rsx + wdx, wdx)
            rsy = jnp.where(same, rsy + wdy, wdy)
            rsz = jnp.where(same, rsz + wdz, wdz)
            rcm = jnp.where(same, rcm + wv, wv)
            rcf = jnp.where(same, rcf + (1.0 - wv), 1.0 - wv)
            return (ss, bv, rsx, rsy, rsz, rcm, rcf)

        carry = lax.fori_loop(0, _BLOCK_A // _L, gbody, carry)

    ss, curb, rsx, rsy, rsz, rcm, rcf = carry
    valid = curb >= 0
    ci = jnp.maximum(curb, 0)
    plsc.addupdate_scatter(acc, [ci], rsx, mask=valid)
    plsc.addupdate_scatter(acc, [_S + ci], rsy, mask=valid)
    plsc.addupdate_scatter(acc, [2 * _S + ci], rsz, mask=valid)
    plsc.addupdate_scatter(acc, [3 * _S + ci], rcm, mask=valid)
    plsc.addupdate_scatter(acc, [4 * _S + ci], rcf, mask=valid)

    ssbuf[...] = ss
    pltpu.sync_copy(acc, part_hbm.at[wid])
    pltpu.sync_copy(ssbuf, ss_hbm.at[wid])


def _tc_merge(part_ref, ss_ref, out_ref):
    p = part_ref[...]
    m = jnp.sum(p, axis=0)
    sx = m[0:1]
    sy = m[1:2]
    sz = m[2:3]
    cm = m[3:4]
    cf = m[4:5]
    s2 = sx * sx + sy * sy + sz * sz
    corr = jnp.sum(jnp.where(cf == 0.0, s2 / jnp.maximum(cm, 1.0), 0.0))
    nm = jnp.sum(cm)
    ssq = jnp.sum(ss_ref[...])
    out_ref[0, 0] = (ssq - corr) / jnp.maximum(nm, 1.0)


def kernel(v_pred, x0, x1, fixed, batch_idx, num_systems):
    n = v_pred.shape[0]
    atoms_per_tile = n // _NW

    vp = v_pred.T.reshape(-1)
    a0f = x0.T.reshape(-1)
    a1f = x1.T.reshape(-1)
    w = (~fixed).astype(jnp.float32)
    bi = batch_idx.astype(jnp.int32)

    fbuf = [pltpu.VMEM((_BLOCK_A,), jnp.float32) for _ in range(10)]
    ibuf = [pltpu.VMEM((_BLOCK_A,), jnp.int32)]
    mesh = plsc.VectorSubcoreMesh(core_axis_name="c", subcore_axis_name="s")
    part, ss = pl.kernel(
        functools.partial(_sc_partials, atoms_per_tile=atoms_per_tile, n=n),
        out_type=(
            jax.ShapeDtypeStruct((_NW, 5 * _S), jnp.float32),
            jax.ShapeDtypeStruct((_NW, _L), jnp.float32),
        ),
        mesh=mesh,
        compiler_params=pltpu.CompilerParams(needs_layout_passes=False),
        scratch_types=(
            tuple(fbuf) + tuple(ibuf),
            tuple(fbuf) + tuple(ibuf),
            pltpu.SemaphoreType.DMA,
            pltpu.SemaphoreType.DMA,
            pltpu.VMEM((5 * _S,), jnp.float32),
            pltpu.VMEM((_L,), jnp.float32),
        ),
    )(vp, a0f, a1f, w, bi)

    out = pl.pallas_call(
        _tc_merge,
        out_shape=jax.ShapeDtypeStruct((1, 1), jnp.float32),
        out_specs=pl.BlockSpec(memory_space=pltpu.SMEM),
    )(part.reshape(_NW, 5, _S), ss)

    loss = out[0, 0]
    return loss + jnp.zeros_like(loss) * num_systems


# speedup vs baseline: 70.7078x; 1.0256x over previous
import functools

import jax
import jax.numpy as jnp
from jax import lax
from jax.experimental import pallas as pl
from jax.experimental.pallas import tpu as pltpu
from jax.experimental.pallas import tpu_sc as plsc

_S = 8192
_NC = 2
_NS = 16
_NW = _NC * _NS
_L = 16
_BLOCK_A = 2048


def _sc_partials(vp_h, x0_h, x1_h, w_h, bi_h,
                 part_hbm, ss_hbm,
                 bufs0, bufs1, sem0, sem1, acc, ssbuf,
                 *, atoms_per_tile, n):
    wid = lax.axis_index("s") * _NC + lax.axis_index("c")
    base_atom = wid * atoms_per_tile
    nblk = atoms_per_tile // _BLOCK_A
    sems = (sem0, sem1)

    zero16 = jnp.zeros((_L,), jnp.float32)

    @plsc.parallel_loop(0, 5 * _S, _L, unroll=8)
    def _zero(o):
        acc[pl.ds(o, _L)] = zero16

    def start_block(blk, parity):
        a0 = base_atom + blk * _BLOCK_A
        bufs, sem = (bufs0, bufs1)[parity], sems[parity]
        hs = []
        for c in range(3):
            hs.append(pltpu.async_copy(
                vp_h.at[pl.ds(c * n + a0, _BLOCK_A)], bufs[c], sem))
            hs.append(pltpu.async_copy(
                x0_h.at[pl.ds(c * n + a0, _BLOCK_A)], bufs[3 + c], sem))
            hs.append(pltpu.async_copy(
                x1_h.at[pl.ds(c * n + a0, _BLOCK_A)], bufs[6 + c], sem))
        hs.append(pltpu.async_copy(w_h.at[pl.ds(a0, _BLOCK_A)], bufs[9], sem))
        hs.append(pltpu.async_copy(bi_h.at[pl.ds(a0, _BLOCK_A)], bufs[10], sem))
        return hs

    pending = {0: start_block(0, 0)}

    ss = jnp.zeros((_L,), jnp.float32)
    curb = jnp.full((_L,), -1, jnp.int32)
    rsx = zero16
    rsy = zero16
    rsz = zero16
    rcm = zero16
    rcf = zero16
    carry = (ss, curb, rsx, rsy, rsz, rcm, rcf)

    for blk in range(nblk):
        parity = blk % 2
        if blk + 1 < nblk:
            pending[(blk + 1) % 2] = start_block(blk + 1, (blk + 1) % 2)
        for h in pending.pop(parity):
            h.wait()
        bufs = (bufs0, bufs1)[parity]
        b0, b1, b2, b3, b4, b5, b6, b7, b8, b9, b10 = bufs

        @plsc.parallel_loop(0, _BLOCK_A, _L, unroll=2, carry=carry)
        def carry(o, carry):
            ss, curb, rsx, rsy, rsz, rcm, rcf = carry
            dx = b0[pl.ds(o, _L)] - b6[pl.ds(o, _L)] + b3[pl.ds(o, _L)]
            dy = b1[pl.ds(o, _L)] - b7[pl.ds(o, _L)] + b4[pl.ds(o, _L)]
            dz = b2[pl.ds(o, _L)] - b8[pl.ds(o, _L)] + b5[pl.ds(o, _L)]
            wv = b9[pl.ds(o, _L)]
            bv = b10[pl.ds(o, _L)]
            wdx = wv * dx
            wdy = wv * dy
            wdz = wv * dz
            ss = ss + wdx * dx + wdy * dy + wdz * dz
            same = bv == curb
            flush = jnp.logical_not(same) & (curb >= 0)
            ci = jnp.maximum(curb, 0)
            plsc.addupdate_scatter(acc, [ci], rsx, mask=flush)
            plsc.addupdate_scatter(acc, [_S + ci], rsy, mask=flush)
            plsc.addupdate_scatter(acc, [2 * _S + ci], rsz, mask=flush)
            plsc.addupdate_scatter(acc, [3 * _S + ci], rcm, mask=flush)
            plsc.addupdate_scatter(acc, [4 * _S + ci], rcf, mask=flush)
            rsx = jnp.where(same, rsx + wdx, wdx)
            rsy = jnp.where(same, rsy + wdy, wdy)
            rsz = jnp.where(same, rsz + wdz, wdz)
            rcm = jnp.where(same, rcm + wv, wv)
            rcf = jnp.where(same, rcf + (1.0 - wv), 1.0 - wv)
            return (ss, bv, rsx, rsy, rsz, rcm, rcf)

    ss, curb, rsx, rsy, rsz, rcm, rcf = carry
    valid = curb >= 0
    ci = jnp.maximum(curb, 0)
    plsc.addupdate_scatter(acc, [ci], rsx, mask=valid)
    plsc.addupdate_scatter(acc, [_S + ci], rsy, mask=valid)
    plsc.addupdate_scatter(acc, [2 * _S + ci], rsz, mask=valid)
    plsc.addupdate_scatter(acc, [3 * _S + ci], rcm, mask=valid)
    plsc.addupdate_scatter(acc, [4 * _S + ci], rcf, mask=valid)

    ssbuf[...] = ss
    pltpu.sync_copy(acc, part_hbm.at[wid])
    pltpu.sync_copy(ssbuf, ss_hbm.at[wid])


def _tc_merge(part_ref, ss_ref, out_ref):
    p = part_ref[...]
    m = jnp.sum(p, axis=0)
    sx = m[0:1]
    sy = m[1:2]
    sz = m[2:3]
    cm = m[3:4]
    cf = m[4:5]
    s2 = sx * sx + sy * sy + sz * sz
    corr = jnp.sum(jnp.where(cf == 0.0, s2 / jnp.maximum(cm, 1.0), 0.0))
    nm = jnp.sum(cm)
    ssq = jnp.sum(ss_ref[...])
    out_ref[0, 0] = (ssq - corr) / jnp.maximum(nm, 1.0)


def kernel(v_pred, x0, x1, fixed, batch_idx, num_systems):
    n = v_pred.shape[0]
    atoms_per_tile = n // _NW

    vp = v_pred.T.reshape(-1)
    a0f = x0.T.reshape(-1)
    a1f = x1.T.reshape(-1)
    w = (~fixed).astype(jnp.float32)
    bi = batch_idx.astype(jnp.int32)

    fbuf = [pltpu.VMEM((_BLOCK_A,), jnp.float32) for _ in range(10)]
    ibuf = [pltpu.VMEM((_BLOCK_A,), jnp.int32)]
    mesh = plsc.VectorSubcoreMesh(core_axis_name="c", subcore_axis_name="s")
    part, ss = pl.kernel(
        functools.partial(_sc_partials, atoms_per_tile=atoms_per_tile, n=n),
        out_type=(
            jax.ShapeDtypeStruct((_NW, 5 * _S), jnp.float32),
            jax.ShapeDtypeStruct((_NW, _L), jnp.float32),
        ),
        mesh=mesh,
        compiler_params=pltpu.CompilerParams(needs_layout_passes=False),
        scratch_types=(
            tuple(fbuf) + tuple(ibuf),
            tuple(fbuf) + tuple(ibuf),
            pltpu.SemaphoreType.DMA,
            pltpu.SemaphoreType.DMA,
            pltpu.VMEM((5 * _S,), jnp.float32),
            pltpu.VMEM((_L,), jnp.float32),
        ),
    )(vp, a0f, a1f, w, bi)

    out = pl.pallas_call(
        _tc_merge,
        out_shape=jax.ShapeDtypeStruct((1, 1), jnp.float32),
        out_specs=pl.BlockSpec(memory_space=pltpu.SMEM),
    )(part.reshape(_NW, 5, _S), ss)

    loss = out[0, 0]
    return loss + jnp.zeros_like(loss) * num_systems
